# baseline (speedup 1.0000x reference)
import functools

import jax
import jax.numpy as jnp
from jax import lax
from jax.experimental import pallas as pl
from jax.experimental.pallas import tpu as pltpu
from jax.experimental.pallas import tpu_sc as plsc

B, C, V_LVL, V_PREV, PATCH = 8, 128, 40962, 10242, 7
ROWS = B * C
NW = 32
PW = 321
G = 6
NBATCH = 54
NIDX = G * PATCH
QLEN = NBATCH * NIDX
QPAD = 2304
FLAT_P = V_PREV * PATCH
PB_CAP = V_PREV - G


@functools.partial(
    pl.kernel,
    out_type=jax.ShapeDtypeStruct((V_PREV, B, C), jnp.float32),
    mesh=plsc.VectorSubcoreMesh(core_axis_name="c", subcore_axis_name="s"),
    compiler_params=pltpu.CompilerParams(
        needs_layout_passes=False, skip_device_barrier=True),
    scratch_types=[
        pltpu.VMEM((2, NIDX, B, C), jnp.float32),
        pltpu.VMEM((2, G, B, C), jnp.float32),
        pltpu.VMEM((NBATCH, NIDX), jnp.int32),
        pltpu.VMEM((QPAD // 128, 128), jnp.int32),
        pltpu.VMEM((QPAD // 128, 128), jnp.int32),
        pltpu.SemaphoreType.DMA,
        pltpu.SemaphoreType.DMA,
        pltpu.SemaphoreType.DMA,
        pltpu.SemaphoreType.DMA,
        pltpu.SemaphoreType.DMA,
        pltpu.SemaphoreType.DMA,
    ],
)
def _sc_maxpool(x_hbm, v2p_hbm, patches_hbm, out_hbm,
                rows_v, obuf_v, comb_v, qbuf_v, vbuf_v,
                sg0, sg1, so0, so1, sc0, sc1):
    wid = lax.axis_index("s") * 2 + lax.axis_index("c")
    p0 = wid * PW
    lane = lax.iota(jnp.int32, 16)

    def pb_of(bi):
        return jnp.minimum(jnp.minimum(p0 + G * bi, p0 + (PW - G)), PB_CAP)

    def fire_gather(slot, sem, bi):
        pltpu.make_async_copy(
            x_hbm.at[comb_v.at[bi]], rows_v.at[slot], sem).start()

    def gather_done(slot, sem):
        pltpu.make_async_copy(
            x_hbm.at[comb_v.at[0]], rows_v.at[slot], sem).wait()

    def fire_out(slot, sem, bi):
        pltpu.make_async_copy(
            obuf_v.at[slot], out_hbm.at[pl.ds(pb_of(bi), G)], sem).start()

    def out_done(slot, sem):
        pltpu.make_async_copy(
            obuf_v.at[slot], out_hbm.at[pl.ds(0, G)], sem).wait()

    def build_q(i, carry):
        f = i * 16 + lane
        bi = f // NIDX
        t = f - bi * NIDX
        p_ = t // PATCH
        j = t - p_ * PATCH
        pb = jnp.minimum(jnp.minimum(p0 + G * bi, p0 + (PW - G)), PB_CAP)
        q = jnp.minimum(j * V_PREV + pb + p_, FLAT_P - 1)
        qbuf_v[i // 8, pl.ds((i % 8) * 16, 16)] = q
        return carry

    lax.fori_loop(0, QPAD // 16, build_q, 0)

    NGRP = QPAD // 128

    def fire_patches(g, carry):
        pltpu.make_async_copy(
            patches_hbm.at[qbuf_v.at[g]], vbuf_v.at[g], sc0).start()
        return carry

    lax.fori_loop(0, NGRP, fire_patches, 0)

    def chain_v2p(g, carry):
        pltpu.make_async_copy(
            patches_hbm.at[qbuf_v.at[0]], vbuf_v.at[g], sc0).wait()
        pltpu.make_async_copy(
            v2p_hbm.at[vbuf_v.at[g]], qbuf_v.at[g], sc1).start()
        return carry

    lax.fori_loop(0, NGRP, chain_v2p, 0)

    def scatter_group(g0, g1, fire_first):
        def scatter_comb(i, carry):
            f = i * 16 + lane
            bi = f // NIDX
            t = f - bi * NIDX
            v = qbuf_v[i // 8, pl.ds((i % 8) * 16, 16)]
            mask = f < QLEN
            plsc.store_scatter(comb_v, [bi, t], v, mask=mask)
            return carry

        def drain_scatter(g, carry):
            pltpu.make_async_copy(
                v2p_hbm.at[vbuf_v.at[0]], qbuf_v.at[g], sc1).wait()
            lax.fori_loop(g * 8, jnp.minimum((g + 1) * 8, (QLEN + 15) // 16),
                          scatter_comb, 0)
            return carry

        lax.fori_loop(g0, g1, drain_scatter, 0)
        if fire_first:
            fire_gather(0, sg0, 0)
            fire_gather(1, sg1, 1)

    scatter_group(0, 1, True)
    scatter_group(1, NGRP, False)

    def compute(slot):
        @plsc.parallel_loop(0, 64, unroll=2)
        def kbody(k):
            kb = k // 8
            c0 = (k - kb * 8) * 16
            for p_ in range(G):
                m = rows_v[slot, PATCH * p_, kb, pl.ds(c0, 16)]
                for j in range(1, PATCH):
                    m = jnp.maximum(
                        m, rows_v[slot, PATCH * p_ + j, kb, pl.ds(c0, 16)])
                obuf_v[slot, p_, kb, pl.ds(c0, 16)] = m

    def step(gg, carry):
        for slot in range(2):
            bi = gg * 2 + slot
            sem = sg0 if slot == 0 else sg1
            osem = so0 if slot == 0 else so1
            gather_done(slot, sem)

            @pl.when(bi >= 2)
            def _():
                out_done(slot, osem)

            compute(slot)

            @pl.when(bi + 2 < NBATCH)
            def _():
                fire_gather(slot, sem, bi + 2)

            fire_out(slot, osem, bi)
        return carry

    lax.fori_loop(0, NBATCH // 2, step, 0)
    out_done(0, so0)
    out_done(1, so1)


def kernel(x, vertices_to_prev_lvl, neihboring_patches):
    x_t = x.transpose(2, 0, 1)
    patches_flat = neihboring_patches.T.reshape(-1)
    out_t = _sc_maxpool(x_t, vertices_to_prev_lvl, patches_flat)
    return out_t.transpose(1, 2, 0)


# speedup vs baseline: 1.0072x; 1.0072x over previous
import functools

import jax
import jax.numpy as jnp
from jax import lax
from jax.experimental import pallas as pl
from jax.experimental.pallas import tpu as pltpu
from jax.experimental.pallas import tpu_sc as plsc

B, C, V_LVL, V_PREV, PATCH = 8, 128, 40962, 10242, 7
ROWS = B * C
NW = 32
PW = 321
G = 6
NBATCH = 54
NIDX = G * PATCH
QLEN = NBATCH * NIDX
QPAD = 2304
FLAT_P = V_PREV * PATCH
PB_CAP = V_PREV - G


@functools.partial(
    pl.kernel,
    out_type=jax.ShapeDtypeStruct((V_PREV, B, C), jnp.float32),
    mesh=plsc.VectorSubcoreMesh(core_axis_name="c", subcore_axis_name="s"),
    compiler_params=pltpu.CompilerParams(
        needs_layout_passes=False, skip_device_barrier=True),
    scratch_types=[
        pltpu.VMEM((2, NIDX, B, C), jnp.float32),
        pltpu.VMEM((2, G, B, C), jnp.float32),
        pltpu.VMEM((NBATCH, NIDX), jnp.int32),
        pltpu.VMEM((QPAD,), jnp.int32),
        pltpu.VMEM((QPAD,), jnp.int32),
        pltpu.SemaphoreType.DMA,
        pltpu.SemaphoreType.DMA,
        pltpu.SemaphoreType.DMA,
        pltpu.SemaphoreType.DMA,
        pltpu.SemaphoreType.DMA,
    ],
)
def _sc_maxpool(x_hbm, v2p_hbm, patches_hbm, out_hbm,
                rows_v, obuf_v, comb_v, qbuf_v, vbuf_v,
                sg0, sg1, so0, so1, sc0):
    wid = lax.axis_index("s") * 2 + lax.axis_index("c")
    p0 = wid * PW
    lane = lax.iota(jnp.int32, 16)

    def pb_of(bi):
        return jnp.minimum(jnp.minimum(p0 + G * bi, p0 + (PW - G)), PB_CAP)

    def fire_gather(slot, sem, bi):
        pltpu.make_async_copy(
            x_hbm.at[comb_v.at[bi]], rows_v.at[slot], sem).start()

    def gather_done(slot, sem):
        pltpu.make_async_copy(
            x_hbm.at[comb_v.at[0]], rows_v.at[slot], sem).wait()

    def fire_out(slot, sem, bi):
        pltpu.make_async_copy(
            obuf_v.at[slot], out_hbm.at[pl.ds(pb_of(bi), G)], sem).start()

    def out_done(slot, sem):
        pltpu.make_async_copy(
            obuf_v.at[slot], out_hbm.at[pl.ds(0, G)], sem).wait()

    def build_q(i, carry):
        f = i * 16 + lane
        bi = f // NIDX
        t = f - bi * NIDX
        p_ = t // PATCH
        j = t - p_ * PATCH
        pb = jnp.minimum(jnp.minimum(p0 + G * bi, p0 + (PW - G)), PB_CAP)
        q = jnp.minimum(j * V_PREV + pb + p_, FLAT_P - 1)
        qbuf_v[pl.ds(i * 16, 16)] = q
        return carry

    lax.fori_loop(0, QPAD // 16, build_q, 0)
    pltpu.make_async_copy(patches_hbm.at[qbuf_v], vbuf_v, sc0).start()
    pltpu.make_async_copy(patches_hbm.at[qbuf_v], vbuf_v, sc0).wait()
    pltpu.make_async_copy(v2p_hbm.at[vbuf_v], qbuf_v, sc0).start()
    pltpu.make_async_copy(v2p_hbm.at[vbuf_v], qbuf_v, sc0).wait()

    def scatter_comb(i, carry):
        f = i * 16 + lane
        bi = f // NIDX
        t = f - bi * NIDX
        v = qbuf_v[pl.ds(i * 16, 16)]
        mask = f < QLEN
        plsc.store_scatter(comb_v, [bi, t], v, mask=mask)
        return carry

    lax.fori_loop(0, (QLEN + 15) // 16, scatter_comb, 0)
    fire_gather(0, sg0, 0)
    fire_gather(1, sg1, 1)

    def compute(slot):
        @plsc.parallel_loop(0, 64, unroll=2)
        def kbody(k):
            kb = k // 8
            c0 = (k - kb * 8) * 16
            for p_ in range(G):
                m = rows_v[slot, PATCH * p_, kb, pl.ds(c0, 16)]
                for j in range(1, PATCH):
                    m = jnp.maximum(
                        m, rows_v[slot, PATCH * p_ + j, kb, pl.ds(c0, 16)])
                obuf_v[slot, p_, kb, pl.ds(c0, 16)] = m

    def step(gg, carry):
        for slot in range(2):
            bi = gg * 2 + slot
            sem = sg0 if slot == 0 else sg1
            osem = so0 if slot == 0 else so1
            gather_done(slot, sem)

            @pl.when(bi >= 2)
            def _():
                out_done(slot, osem)

            compute(slot)

            @pl.when(bi + 2 < NBATCH)
            def _():
                fire_gather(slot, sem, bi + 2)

            fire_out(slot, osem, bi)
        return carry

    lax.fori_loop(0, NBATCH // 2, step, 0)
    out_done(0, so0)
    out_done(1, so1)


def kernel(x, vertices_to_prev_lvl, neihboring_patches):
    x_t = x.transpose(2, 0, 1)
    patches_flat = neihboring_patches.T.reshape(-1)
    out_t = _sc_maxpool(x_t, vertices_to_prev_lvl, patches_flat)
    return out_t.transpose(1, 2, 0)


# speedup vs baseline: 1.0189x; 1.0115x over previous
import functools

import jax
import jax.numpy as jnp
from jax import lax
from jax.experimental import pallas as pl
from jax.experimental.pallas import tpu as pltpu
from jax.experimental.pallas import tpu_sc as plsc

B, C, V_LVL, V_PREV, PATCH = 8, 128, 40962, 10242, 7
ROWS = B * C
NW = 32
PW = 321
G = 7
NBATCH = 46
NIDX = G * PATCH
QLEN = NBATCH * NIDX
QPAD = 2256
FLAT_P = V_PREV * PATCH
PB_CAP = V_PREV - G


@functools.partial(
    pl.kernel,
    out_type=jax.ShapeDtypeStruct((V_PREV, B, C), jnp.float32),
    mesh=plsc.VectorSubcoreMesh(core_axis_name="c", subcore_axis_name="s"),
    compiler_params=pltpu.CompilerParams(
        needs_layout_passes=False, skip_device_barrier=True),
    scratch_types=[
        pltpu.VMEM((2, NIDX, B, C), jnp.float32),
        pltpu.VMEM((2, G, B, C), jnp.float32),
        pltpu.VMEM((NBATCH, NIDX), jnp.int32),
        pltpu.VMEM((QPAD,), jnp.int32),
        pltpu.VMEM((QPAD,), jnp.int32),
        pltpu.SemaphoreType.DMA,
        pltpu.SemaphoreType.DMA,
        pltpu.SemaphoreType.DMA,
        pltpu.SemaphoreType.DMA,
        pltpu.SemaphoreType.DMA,
    ],
)
def _sc_maxpool(x_hbm, v2p_hbm, patches_hbm, out_hbm,
                rows_v, obuf_v, comb_v, qbuf_v, vbuf_v,
                sg0, sg1, so0, so1, sc0):
    wid = lax.axis_index("s") * 2 + lax.axis_index("c")
    p0 = wid * PW
    lane = lax.iota(jnp.int32, 16)

    def pb_of(bi):
        return jnp.minimum(jnp.minimum(p0 + G * bi, p0 + (PW - G)), PB_CAP)

    def fire_gather(slot, sem, bi):
        pltpu.make_async_copy(
            x_hbm.at[comb_v.at[bi]], rows_v.at[slot], sem).start()

    def gather_done(slot, sem):
        pltpu.make_async_copy(
            x_hbm.at[comb_v.at[0]], rows_v.at[slot], sem).wait()

    def fire_out(slot, sem, bi):
        pltpu.make_async_copy(
            obuf_v.at[slot], out_hbm.at[pl.ds(pb_of(bi), G)], sem).start()

    def out_done(slot, sem):
        pltpu.make_async_copy(
            obuf_v.at[slot], out_hbm.at[pl.ds(0, G)], sem).wait()

    def build_q(i, carry):
        f = i * 16 + lane
        bi = f // NIDX
        t = f - bi * NIDX
        p_ = t // PATCH
        j = t - p_ * PATCH
        pb = jnp.minimum(jnp.minimum(p0 + G * bi, p0 + (PW - G)), PB_CAP)
        q = jnp.minimum(j * V_PREV + pb + p_, FLAT_P - 1)
        qbuf_v[pl.ds(i * 16, 16)] = q
        return carry

    lax.fori_loop(0, QPAD // 16, build_q, 0)
    pltpu.make_async_copy(patches_hbm.at[qbuf_v], vbuf_v, sc0).start()
    pltpu.make_async_copy(patches_hbm.at[qbuf_v], vbuf_v, sc0).wait()
    pltpu.make_async_copy(v2p_hbm.at[vbuf_v], qbuf_v, sc0).start()
    pltpu.make_async_copy(v2p_hbm.at[vbuf_v], qbuf_v, sc0).wait()

    def scatter_comb(i, carry):
        f = i * 16 + lane
        bi = f // NIDX
        t = f - bi * NIDX
        v = qbuf_v[pl.ds(i * 16, 16)]
        mask = f < QLEN
        plsc.store_scatter(comb_v, [bi, t], v, mask=mask)
        return carry

    lax.fori_loop(0, (QLEN + 15) // 16, scatter_comb, 0)
    fire_gather(0, sg0, 0)
    fire_gather(1, sg1, 1)

    def compute(slot):
        @plsc.parallel_loop(0, 64, unroll=2)
        def kbody(k):
            kb = k // 8
            c0 = (k - kb * 8) * 16
            for p_ in range(G):
                m = rows_v[slot, PATCH * p_, kb, pl.ds(c0, 16)]
                for j in range(1, PATCH):
                    m = jnp.maximum(
                        m, rows_v[slot, PATCH * p_ + j, kb, pl.ds(c0, 16)])
                obuf_v[slot, p_, kb, pl.ds(c0, 16)] = m

    def step(gg, carry):
        for slot in range(2):
            bi = gg * 2 + slot
            sem = sg0 if slot == 0 else sg1
            osem = so0 if slot == 0 else so1
            gather_done(slot, sem)

            @pl.when(bi >= 2)
            def _():
                out_done(slot, osem)

            compute(slot)

            @pl.when(bi + 2 < NBATCH)
            def _():
                fire_gather(slot, sem, bi + 2)

            fire_out(slot, osem, bi)
        return carry

    lax.fori_loop(0, NBATCH // 2, step, 0)
    out_done(0, so0)
    out_done(1, so1)


def kernel(x, vertices_to_prev_lvl, neihboring_patches):
    x_t = x.transpose(2, 0, 1)
    patches_flat = neihboring_patches.T.reshape(-1)
    out_t = _sc_maxpool(x_t, vertices_to_prev_lvl, patches_flat)
    return out_t.transpose(1, 2, 0)
